# Initial kernel scaffold; baseline (speedup 1.0000x reference)
import jax
import jax.numpy as jnp
from jax.experimental import pallas as pl


def kernel(x, edge_index, Wg, a_src, a_dst, Wz, Uz, bz, Wr, Ur, br, Wh, Uh, bh):
    raise NotImplementedError("write your pallas kernel here")


# speedup vs baseline: 1.4200x; 1.4200x over previous
import functools

import jax
import jax.numpy as jnp
from jax.experimental import pallas as pl
from jax.experimental.pallas import tpu as pltpu

N = 10000
E = 320000
H = 128
N_BLK = 400


def _k0_body(x_ref, wz_ref, wr_ref, wh_ref, bz_ref, br_ref, bh_ref,
             xz_ref, xr_ref, xh_ref, h1_ref):
    x = x_ref[...]
    xz = x @ wz_ref[...] + bz_ref[...]
    xr = x @ wr_ref[...] + br_ref[...]
    xh = x @ wh_ref[...] + bh_ref[...]
    xz_ref[...] = xz
    xr_ref[...] = xr
    xh_ref[...] = xh
    h1_ref[...] = jax.nn.sigmoid(xz) * jnp.tanh(xh)


def _k1_body(h_ref, wg_ref, asrc_ref, adst_ref, hw_ref, s_ref, t_ref):
    hw = h_ref[...] @ wg_ref[...]
    hw_ref[...] = hw
    s_ref[...] = hw @ asrc_ref[...]
    t_ref[...] = hw @ adst_ref[...]


def _k2_body(agg_ref, xz_ref, xr_ref, xh_ref, uz_ref, ur_ref, uh_ref,
             out_ref):
    agg = agg_ref[...]
    z = jax.nn.sigmoid(xz_ref[...] + agg @ uz_ref[...])
    r = jax.nn.sigmoid(xr_ref[...] + agg @ ur_ref[...])
    htil = jnp.tanh(xh_ref[...] + (r * agg) @ uh_ref[...])
    out_ref[...] = (1.0 - z) * agg + z * htil


def _row_spec():
    return pl.BlockSpec((N_BLK, H), lambda i: (i, 0))


def _full_spec(shape):
    return pl.BlockSpec(shape, lambda i: tuple(0 for _ in shape))


_GRID = (N // N_BLK,)


@jax.jit
def _k0(x, wz, wr, wh, bz, br, bh):
    f = pl.pallas_call(
        _k0_body,
        grid=_GRID,
        in_specs=[_row_spec()] + [_full_spec((H, H))] * 3
        + [_full_spec((1, H))] * 3,
        out_specs=[_row_spec()] * 4,
        out_shape=[jax.ShapeDtypeStruct((N, H), jnp.float32)] * 4,
    )
    return f(x, wz, wr, wh, bz.reshape(1, H), br.reshape(1, H),
             bh.reshape(1, H))


@jax.jit
def _k1(h, wg, a_src, a_dst):
    f = pl.pallas_call(
        _k1_body,
        grid=_GRID,
        in_specs=[_row_spec(), _full_spec((H, H)), _full_spec((H, 1)),
                  _full_spec((H, 1))],
        out_specs=[_row_spec(), pl.BlockSpec((N_BLK, 1), lambda i: (i, 0)),
                   pl.BlockSpec((N_BLK, 1), lambda i: (i, 0))],
        out_shape=[jax.ShapeDtypeStruct((N, H), jnp.float32),
                   jax.ShapeDtypeStruct((N, 1), jnp.float32),
                   jax.ShapeDtypeStruct((N, 1), jnp.float32)],
    )
    return f(h, wg, a_src.reshape(H, 1), a_dst.reshape(H, 1))


@jax.jit
def _k2(agg, xz, xr, xh, uz, ur, uh):
    f = pl.pallas_call(
        _k2_body,
        grid=_GRID,
        in_specs=[_row_spec()] * 4 + [_full_spec((H, H))] * 3,
        out_specs=_row_spec(),
        out_shape=jax.ShapeDtypeStruct((N, H), jnp.float32),
    )
    return f(agg, xz, xr, xh, uz, ur, uh)


def _edge_pass(hw, s, t, src, dst):
    e = jax.nn.leaky_relu(s[src] + t[dst], 0.2)
    ex = jnp.exp(e)
    denom = jax.ops.segment_sum(ex, dst, num_segments=N)
    alpha = ex / denom[dst]
    m = jax.ops.segment_max(alpha[:, None] * hw[src], dst, num_segments=N)
    return jnp.where(jnp.isfinite(m), jnp.where(m > 0, m, jnp.exp(m) - 1.0),
                     0.0)


def kernel(x, edge_index, Wg, a_src, a_dst, Wz, Uz, bz, Wr, Ur, br,
           Wh, Uh, bh):
    src = edge_index[0]
    dst = edge_index[1]
    xz, xr, xh, h = _k0(x, Wz, Wr, Wh, bz, br, bh)
    for _ in range(2):
        hw, s, t = _k1(h, Wg, a_src, a_dst)
        agg = _edge_pass(hw, s[:, 0], t[:, 0], src, dst)
        h = _k2(agg, xz, xr, xh, Uz, Ur, Uh)
    return h


# speedup vs baseline: 9.9663x; 7.0183x over previous
import functools

import jax
import jax.numpy as jnp
from jax import lax
from jax.experimental import pallas as pl
from jax.experimental.pallas import tpu as pltpu
from jax.experimental.pallas import tpu_sc as plsc

N = 10000
E = 320000
H = 128
N_BLK = 400

NW = 32
NPT = 320
N_PAD = NW * NPT
ECAP = 12288
KR = 128
CE = 4096
E_PAD = ((E + CE - 1) // CE) * CE
NEG = float("-inf")


def _k0_body(x_ref, wz_ref, wr_ref, wh_ref, bz_ref, br_ref, bh_ref,
             xz_ref, xr_ref, xh_ref, h1_ref):
    x = x_ref[...]
    xz = x @ wz_ref[...] + bz_ref[...]
    xr = x @ wr_ref[...] + br_ref[...]
    xh = x @ wh_ref[...] + bh_ref[...]
    xz_ref[...] = xz
    xr_ref[...] = xr
    xh_ref[...] = xh
    h1_ref[...] = jax.nn.sigmoid(xz) * jnp.tanh(xh)


def _k1_body(h_ref, wg_ref, asrc_ref, adst_ref, hw_ref, s_ref, t_ref):
    hw = h_ref[...] @ wg_ref[...]
    hw_ref[...] = hw
    s_ref[...] = hw @ asrc_ref[...]
    t_ref[...] = hw @ adst_ref[...]


def _k2_body(agg_ref, xz_ref, xr_ref, xh_ref, uz_ref, ur_ref, uh_ref,
             out_ref):
    agg = agg_ref[...]
    z = jax.nn.sigmoid(xz_ref[...] + agg @ uz_ref[...])
    r = jax.nn.sigmoid(xr_ref[...] + agg @ ur_ref[...])
    htil = jnp.tanh(xh_ref[...] + (r * agg) @ uh_ref[...])
    out_ref[...] = (1.0 - z) * agg + z * htil


def _row_spec():
    return pl.BlockSpec((N_BLK, H), lambda i: (i, 0))


def _full_spec(shape):
    return pl.BlockSpec(shape, lambda i: tuple(0 for _ in shape))


_GRID = (N // N_BLK,)


@jax.jit
def _k0(x, wz, wr, wh, bz, br, bh):
    f = pl.pallas_call(
        _k0_body,
        grid=_GRID,
        in_specs=[_row_spec()] + [_full_spec((H, H))] * 3
        + [_full_spec((1, H))] * 3,
        out_specs=[_row_spec()] * 4,
        out_shape=[jax.ShapeDtypeStruct((N, H), jnp.float32)] * 4,
    )
    return f(x, wz, wr, wh, bz.reshape(1, H), br.reshape(1, H),
             bh.reshape(1, H))


@jax.jit
def _k1(h, wg, a_src, a_dst):
    f = pl.pallas_call(
        _k1_body,
        grid=_GRID,
        in_specs=[_row_spec(), _full_spec((H, H)), _full_spec((H, 1)),
                  _full_spec((H, 1))],
        out_specs=[_row_spec(), pl.BlockSpec((N_BLK, 1), lambda i: (i, 0)),
                   pl.BlockSpec((N_BLK, 1), lambda i: (i, 0))],
        out_shape=[jax.ShapeDtypeStruct((N, H), jnp.float32),
                   jax.ShapeDtypeStruct((N, 1), jnp.float32),
                   jax.ShapeDtypeStruct((N, 1), jnp.float32)],
    )
    return f(h, wg, a_src.reshape(H, 1), a_dst.reshape(H, 1))


@jax.jit
def _k2(agg, xz, xr, xh, uz, ur, uh):
    f = pl.pallas_call(
        _k2_body,
        grid=_GRID,
        in_specs=[_row_spec()] * 4 + [_full_spec((H, H))] * 3,
        out_specs=_row_spec(),
        out_shape=jax.ShapeDtypeStruct((N, H), jnp.float32),
    )
    return f(agg, xz, xr, xh, uz, ur, uh)


def _sc_bin_body(src_hbm, dst_hbm, srcb_hbm, dstb_hbm, cnt_hbm,
                 sbuf, dbuf, src_loc, dst_loc, cnt16_v):
    wid = lax.axis_index("s") * 2 + lax.axis_index("c")
    n0 = wid * NPT

    def memset_body(i, _):
        src_loc[pl.ds(i * 16, 16)] = jnp.zeros((16,), jnp.int32)
        dst_loc[pl.ds(i * 16, 16)] = jnp.full((16,), N_PAD, jnp.int32)
        return 0
    lax.fori_loop(0, (ECAP + 16) // 16, memset_body, 0)

    def chunk_body(c, cnt):
        pltpu.sync_copy(src_hbm.at[pl.ds(c * CE, CE)], sbuf)
        pltpu.sync_copy(dst_hbm.at[pl.ds(c * CE, CE)], dbuf)

        def g_body(g, cnt):
            sl = pl.ds(g * 16, 16)
            d16 = dbuf[sl]
            s16 = sbuf[sl]
            m = (d16 >= n0) & (d16 < n0 + NPT)
            plsc.store_compressed(src_loc.at[pl.ds(cnt, 16)], s16, mask=m)
            plsc.store_compressed(dst_loc.at[pl.ds(cnt, 16)], d16, mask=m)
            return cnt + plsc.all_reduce_population_count(m)[0]
        return lax.fori_loop(0, CE // 16, g_body, cnt)
    cnt = lax.fori_loop(0, E_PAD // CE, chunk_body, 0)

    cnt16_v[...] = jnp.full((16,), cnt, jnp.int32)
    pltpu.sync_copy(cnt16_v, cnt_hbm.at[pl.ds(wid * 16, 16)])
    pltpu.sync_copy(src_loc.at[pl.ds(0, ECAP)],
                    srcb_hbm.at[pl.ds(wid * ECAP, ECAP)])
    pltpu.sync_copy(dst_loc.at[pl.ds(0, ECAP)],
                    dstb_hbm.at[pl.ds(wid * ECAP, ECAP)])


@jax.jit
def _sc_bin(src_p, dst_p):
    mesh = plsc.VectorSubcoreMesh(core_axis_name="c", subcore_axis_name="s")
    f = pl.kernel(
        _sc_bin_body,
        mesh=mesh,
        compiler_params=pltpu.CompilerParams(needs_layout_passes=False),
        out_type=[jax.ShapeDtypeStruct((NW * ECAP,), jnp.int32),
                  jax.ShapeDtypeStruct((NW * ECAP,), jnp.int32),
                  jax.ShapeDtypeStruct((NW * 16,), jnp.int32)],
        scratch_types=[
            pltpu.VMEM((CE,), jnp.int32),
            pltpu.VMEM((CE,), jnp.int32),
            pltpu.VMEM((ECAP + 16,), jnp.int32),
            pltpu.VMEM((ECAP + 16,), jnp.int32),
            pltpu.VMEM((16,), jnp.int32),
        ],
    )
    return f(src_p, dst_p)


def _sc_edge_body(hw_hbm, s_hbm, t_hbm, srcb_hbm, dstb_hbm, cnt_hbm, out_hbm,
                  s_v, t_v, dinv_v, cnt16_v, srcs_v, dstg_v, alpha_v,
                  rows0_v, rows1_v, agg_v, sem0, sem1):
    wid = lax.axis_index("s") * 2 + lax.axis_index("c")
    n0 = wid * NPT

    pltpu.sync_copy(s_hbm, s_v)
    pltpu.sync_copy(t_hbm.at[pl.ds(n0, NPT)], t_v)
    pltpu.sync_copy(cnt_hbm.at[pl.ds(wid * 16, 16)], cnt16_v)
    cnt = cnt16_v[...][0]
    pltpu.sync_copy(srcb_hbm.at[pl.ds(wid * ECAP, ECAP)],
                    srcs_v.at[pl.ds(0, ECAP)])
    pltpu.sync_copy(dstb_hbm.at[pl.ds(wid * ECAP, ECAP)],
                    dstg_v.at[pl.ds(0, ECAP)])

    lanes = lax.iota(jnp.int32, 16)
    ngrp = (cnt + 15) // 16

    def zero_body(i, _):
        dinv_v[pl.ds(i * 16, 16)] = jnp.zeros((16,), jnp.float32)
        return 0
    lax.fori_loop(0, NPT // 16, zero_body, 0)

    def agg_init(d, _):
        for r in range(8):
            agg_v[d, pl.ds(r * 16, 16)] = jnp.full((16,), NEG, jnp.float32)
        return 0
    lax.fori_loop(0, NPT, agg_init, 0)

    def p1_body(g, _):
        sl = pl.ds(g * 16, 16)
        valid = (g * 16 + lanes) < cnt
        srcv = srcs_v[sl]
        dlv = jnp.where(valid, dstg_v[sl] - n0, 0)
        sg = plsc.load_gather(s_v, [srcv])
        tg = plsc.load_gather(t_v, [dlv])
        e = sg + tg
        e = jnp.where(e >= 0.0, e, 0.2 * e)
        w = jnp.where(valid, jnp.exp(e), 0.0)
        alpha_v[sl] = w
        plsc.addupdate_scatter(dinv_v, [dlv], w)
        return 0
    lax.fori_loop(0, ngrp, p1_body, 0)

    def inv_body(i, _):
        sl = pl.ds(i * 16, 16)
        dinv_v[sl] = 1.0 / dinv_v[sl]
        return 0
    lax.fori_loop(0, NPT // 16, inv_body, 0)

    def a_body(g, _):
        sl = pl.ds(g * 16, 16)
        valid = (g * 16 + lanes) < cnt
        dlv = jnp.where(valid, dstg_v[sl] - n0, 0)
        alpha_v[sl] = alpha_v[sl] * plsc.load_gather(dinv_v, [dlv])
        return 0
    lax.fori_loop(0, ngrp, a_body, 0)

    nch = (cnt + KR - 1) // KR

    def start_gather(c, buf, sem):
        pltpu.make_async_copy(
            hw_hbm.at[srcs_v.at[pl.ds(c * KR, KR)]], buf, sem).start()

    def wait_gather(c, buf, sem):
        pltpu.make_async_copy(
            hw_hbm.at[srcs_v.at[pl.ds(c * KR, KR)]], buf, sem).wait()

    def consume(c, buf):
        def e_body(j, _):
            pos = c * KR + j
            valid = pos < cnt
            dl = jnp.where(valid, dstg_v[pl.ds(pos, 16)][0] - n0, 0)
            ab = jnp.full((16,), alpha_v[pl.ds(pos, 16)][0], jnp.float32)
            vb = jnp.full((16,), valid, jnp.bool_)
            for r in range(8):
                sl = pl.ds(r * 16, 16)
                rowv = buf[j, sl]
                aggv = agg_v[dl, sl]
                newv = jnp.maximum(aggv, ab * rowv)
                agg_v[dl, sl] = jnp.where(vb, newv, aggv)
            return 0
        lax.fori_loop(0, KR, e_body, 0)

    def chunk_body(c, _):
        start_gather(c, rows0_v, sem0)
        wait_gather(c, rows0_v, sem0)
        consume(c, rows0_v)
        return 0
    lax.fori_loop(0, nch, chunk_body, 0)
    del rows1_v, sem1

    def fin_body(d, _):
        for r in range(8):
            sl = pl.ds(r * 16, 16)
            v = agg_v[d, sl]
            elu = jnp.where(v > 0.0, v, jnp.exp(v) - 1.0)
            agg_v[d, sl] = jnp.where(v < -3e38, 0.0, elu)
        return 0
    lax.fori_loop(0, NPT, fin_body, 0)
    pltpu.sync_copy(agg_v, out_hbm.at[pl.ds(n0, NPT)])


@jax.jit
def _sc_edge(hw, s_pad, t_pad, srcb, dstb, cnt):
    mesh = plsc.VectorSubcoreMesh(core_axis_name="c", subcore_axis_name="s")
    f = pl.kernel(
        _sc_edge_body,
        mesh=mesh,
        compiler_params=pltpu.CompilerParams(needs_layout_passes=False),
        out_type=jax.ShapeDtypeStruct((N_PAD, H), jnp.float32),
        scratch_types=[
            pltpu.VMEM((N_PAD,), jnp.float32),
            pltpu.VMEM((NPT,), jnp.float32),
            pltpu.VMEM((NPT,), jnp.float32),
            pltpu.VMEM((16,), jnp.int32),
            pltpu.VMEM((ECAP + 16,), jnp.int32),
            pltpu.VMEM((ECAP + 16,), jnp.int32),
            pltpu.VMEM((ECAP + 16,), jnp.float32),
            pltpu.VMEM((KR, H), jnp.float32),
            pltpu.VMEM((KR, H), jnp.float32),
            pltpu.VMEM((NPT, H), jnp.float32),
            pltpu.SemaphoreType.DMA,
            pltpu.SemaphoreType.DMA,
        ],
    )
    return f(hw, s_pad, t_pad, srcb, dstb, cnt)


def kernel(x, edge_index, Wg, a_src, a_dst, Wz, Uz, bz, Wr, Ur, br,
           Wh, Uh, bh):
    src = edge_index[0]
    dst = edge_index[1]
    src_p = jnp.pad(src, (0, E_PAD - E))
    dst_p = jnp.pad(dst, (0, E_PAD - E), constant_values=N_PAD)
    srcb, dstb, cnt = _sc_bin(src_p, dst_p)

    xz, xr, xh, h = _k0(x, Wz, Wr, Wh, bz, br, bh)
    for _ in range(2):
        hw, s, t = _k1(h, Wg, a_src, a_dst)
        s_pad = jnp.pad(s[:, 0], (0, N_PAD - N))
        t_pad = jnp.pad(t[:, 0], (0, N_PAD - N))
        agg = _sc_edge(hw, s_pad, t_pad, srcb, dstb, cnt)[:N]
        h = _k2(agg, xz, xr, xh, Uz, Ur, Uh)
    return h


# speedup vs baseline: 10.7138x; 1.0750x over previous
import functools

import jax
import jax.numpy as jnp
from jax import lax
from jax.experimental import pallas as pl
from jax.experimental.pallas import tpu as pltpu
from jax.experimental.pallas import tpu_sc as plsc

N = 10000
E = 320000
H = 128
N_BLK = 400

NW = 32
NPT = 320
N_PAD = NW * NPT
ECAP = 12288
KR = 128
CE = 4096
E_PAD = ((E + CE - 1) // CE) * CE
NEG = float("-inf")


def _k0_body(x_ref, wz_ref, wr_ref, wh_ref, bz_ref, br_ref, bh_ref,
             xz_ref, xr_ref, xh_ref, h1_ref):
    x = x_ref[...]
    xz = x @ wz_ref[...] + bz_ref[...]
    xr = x @ wr_ref[...] + br_ref[...]
    xh = x @ wh_ref[...] + bh_ref[...]
    xz_ref[...] = xz
    xr_ref[...] = xr
    xh_ref[...] = xh
    h1_ref[...] = jax.nn.sigmoid(xz) * jnp.tanh(xh)


def _k1_body(h_ref, wg_ref, asrc_ref, adst_ref, hw_ref, s_ref, t_ref):
    hw = h_ref[...] @ wg_ref[...]
    hw_ref[...] = hw
    s_ref[...] = hw @ asrc_ref[...]
    t_ref[...] = hw @ adst_ref[...]


def _k2_body(agg_ref, xz_ref, xr_ref, xh_ref, uz_ref, ur_ref, uh_ref,
             out_ref):
    agg = agg_ref[...]
    z = jax.nn.sigmoid(xz_ref[...] + agg @ uz_ref[...])
    r = jax.nn.sigmoid(xr_ref[...] + agg @ ur_ref[...])
    htil = jnp.tanh(xh_ref[...] + (r * agg) @ uh_ref[...])
    out_ref[...] = (1.0 - z) * agg + z * htil


def _row_spec():
    return pl.BlockSpec((N_BLK, H), lambda i: (i, 0))


def _full_spec(shape):
    return pl.BlockSpec(shape, lambda i: tuple(0 for _ in shape))


_GRID = (N // N_BLK,)


@jax.jit
def _k0(x, wz, wr, wh, bz, br, bh):
    f = pl.pallas_call(
        _k0_body,
        grid=_GRID,
        in_specs=[_row_spec()] + [_full_spec((H, H))] * 3
        + [_full_spec((1, H))] * 3,
        out_specs=[_row_spec()] * 4,
        out_shape=[jax.ShapeDtypeStruct((N, H), jnp.float32)] * 4,
    )
    return f(x, wz, wr, wh, bz.reshape(1, H), br.reshape(1, H),
             bh.reshape(1, H))


@jax.jit
def _k1(h, wg, a_src, a_dst):
    f = pl.pallas_call(
        _k1_body,
        grid=_GRID,
        in_specs=[_row_spec(), _full_spec((H, H)), _full_spec((H, 1)),
                  _full_spec((H, 1))],
        out_specs=[_row_spec(), pl.BlockSpec((N_BLK, 1), lambda i: (i, 0)),
                   pl.BlockSpec((N_BLK, 1), lambda i: (i, 0))],
        out_shape=[jax.ShapeDtypeStruct((N, H), jnp.float32),
                   jax.ShapeDtypeStruct((N, 1), jnp.float32),
                   jax.ShapeDtypeStruct((N, 1), jnp.float32)],
    )
    return f(h, wg, a_src.reshape(H, 1), a_dst.reshape(H, 1))


@jax.jit
def _k2(agg, xz, xr, xh, uz, ur, uh):
    f = pl.pallas_call(
        _k2_body,
        grid=_GRID,
        in_specs=[_row_spec()] * 4 + [_full_spec((H, H))] * 3,
        out_specs=_row_spec(),
        out_shape=jax.ShapeDtypeStruct((N, H), jnp.float32),
    )
    return f(agg, xz, xr, xh, uz, ur, uh)


def _sc_bin_body(src_hbm, dst_hbm, srcb_hbm, dstb_hbm, cnt_hbm,
                 sbuf0, dbuf0, sbuf1, dbuf1, src_loc, dst_loc, cnt16_v,
                 semA, semB):
    wid = lax.axis_index("s") * 2 + lax.axis_index("c")
    n0 = wid * NPT

    def memset_body(i, _):
        src_loc[pl.ds(i * 16, 16)] = jnp.zeros((16,), jnp.int32)
        dst_loc[pl.ds(i * 16, 16)] = jnp.full((16,), N_PAD, jnp.int32)
        return 0
    lax.fori_loop(0, (ECAP + 16) // 16, memset_body, 0)

    def start_pair(c, sb, db, sem):
        pltpu.make_async_copy(src_hbm.at[pl.ds(c * CE, CE)], sb, sem).start()
        pltpu.make_async_copy(dst_hbm.at[pl.ds(c * CE, CE)], db, sem).start()

    def wait_pair(c, sb, db, sem):
        pltpu.make_async_copy(src_hbm.at[pl.ds(c * CE, CE)], sb, sem).wait()
        pltpu.make_async_copy(dst_hbm.at[pl.ds(c * CE, CE)], db, sem).wait()

    def scan(sb, db, cnt):
        def g_body(g, cnt):
            sl = pl.ds(g * 16, 16)
            d16 = db[sl]
            s16 = sb[sl]
            m = (d16 >= n0) & (d16 < n0 + NPT)
            plsc.store_compressed(src_loc.at[pl.ds(cnt, 16)], s16, mask=m)
            plsc.store_compressed(dst_loc.at[pl.ds(cnt, 16)], d16, mask=m)
            return cnt + plsc.all_reduce_population_count(m)[0]
        return lax.fori_loop(0, CE // 16, g_body, cnt)

    start_pair(0, sbuf0, dbuf0, semA)

    def pbody(p, cnt):
        c0 = 2 * p
        start_pair(c0 + 1, sbuf1, dbuf1, semB)
        wait_pair(c0, sbuf0, dbuf0, semA)
        cnt = scan(sbuf0, dbuf0, cnt)
        start_pair(c0 + 2, sbuf0, dbuf0, semA)
        wait_pair(c0 + 1, sbuf1, dbuf1, semB)
        return scan(sbuf1, dbuf1, cnt)
    cnt = lax.fori_loop(0, (E_PAD // CE) // 2, pbody, 0)
    wait_pair(E_PAD // CE - 1, sbuf0, dbuf0, semA)
    cnt = scan(sbuf0, dbuf0, cnt)

    cnt16_v[...] = jnp.full((16,), cnt, jnp.int32)
    pltpu.sync_copy(cnt16_v, cnt_hbm.at[pl.ds(wid * 16, 16)])
    pltpu.sync_copy(src_loc.at[pl.ds(0, ECAP)],
                    srcb_hbm.at[pl.ds(wid * ECAP, ECAP)])
    pltpu.sync_copy(dst_loc.at[pl.ds(0, ECAP)],
                    dstb_hbm.at[pl.ds(wid * ECAP, ECAP)])


@jax.jit
def _sc_bin(src_p, dst_p):
    mesh = plsc.VectorSubcoreMesh(core_axis_name="c", subcore_axis_name="s")
    f = pl.kernel(
        _sc_bin_body,
        mesh=mesh,
        compiler_params=pltpu.CompilerParams(needs_layout_passes=False),
        out_type=[jax.ShapeDtypeStruct((NW * ECAP,), jnp.int32),
                  jax.ShapeDtypeStruct((NW * ECAP,), jnp.int32),
                  jax.ShapeDtypeStruct((NW * 16,), jnp.int32)],
        scratch_types=[
            pltpu.VMEM((CE,), jnp.int32),
            pltpu.VMEM((CE,), jnp.int32),
            pltpu.VMEM((CE,), jnp.int32),
            pltpu.VMEM((CE,), jnp.int32),
            pltpu.VMEM((ECAP + 16,), jnp.int32),
            pltpu.VMEM((ECAP + 16,), jnp.int32),
            pltpu.VMEM((16,), jnp.int32),
            pltpu.SemaphoreType.DMA,
            pltpu.SemaphoreType.DMA,
        ],
    )
    return f(src_p, dst_p)


def _sc_edge_body(hw_hbm, s_hbm, t_hbm, srcb_hbm, dstb_hbm, cnt_hbm, out_hbm,
                  s_v, t_v, dinv_v, cnt16_v, srcs_v, dstg_v, alpha_v,
                  rows0_v, rows1_v, agg_v, sem0, sem1):
    wid = lax.axis_index("s") * 2 + lax.axis_index("c")
    n0 = wid * NPT

    pltpu.sync_copy(s_hbm, s_v)
    pltpu.sync_copy(t_hbm.at[pl.ds(n0, NPT)], t_v)
    pltpu.sync_copy(cnt_hbm.at[pl.ds(wid * 16, 16)], cnt16_v)
    cnt = cnt16_v[...][0]
    pltpu.sync_copy(srcb_hbm.at[pl.ds(wid * ECAP, ECAP)],
                    srcs_v.at[pl.ds(0, ECAP)])
    pltpu.sync_copy(dstb_hbm.at[pl.ds(wid * ECAP, ECAP)],
                    dstg_v.at[pl.ds(0, ECAP)])

    lanes = lax.iota(jnp.int32, 16)
    ngrp = (cnt + 15) // 16

    def zero_body(i, _):
        dinv_v[pl.ds(i * 16, 16)] = jnp.zeros((16,), jnp.float32)
        return 0
    lax.fori_loop(0, NPT // 16, zero_body, 0)

    def agg_init(d, _):
        for r in range(8):
            agg_v[d, pl.ds(r * 16, 16)] = jnp.full((16,), NEG, jnp.float32)
        return 0
    lax.fori_loop(0, NPT, agg_init, 0)

    def p1_body(g, _):
        sl = pl.ds(g * 16, 16)
        valid = (g * 16 + lanes) < cnt
        srcv = srcs_v[sl]
        dlv = jnp.where(valid, dstg_v[sl] - n0, 0)
        sg = plsc.load_gather(s_v, [srcv])
        tg = plsc.load_gather(t_v, [dlv])
        e = sg + tg
        e = jnp.where(e >= 0.0, e, 0.2 * e)
        w = jnp.where(valid, jnp.exp(e), 0.0)
        alpha_v[sl] = w
        plsc.addupdate_scatter(dinv_v, [dlv], w)
        return 0
    lax.fori_loop(0, ngrp, p1_body, 0)

    def inv_body(i, _):
        sl = pl.ds(i * 16, 16)
        dinv_v[sl] = 1.0 / dinv_v[sl]
        return 0
    lax.fori_loop(0, NPT // 16, inv_body, 0)

    def a_body(g, _):
        sl = pl.ds(g * 16, 16)
        valid = (g * 16 + lanes) < cnt
        dlv = jnp.where(valid, dstg_v[sl] - n0, 0)
        alpha_v[sl] = alpha_v[sl] * plsc.load_gather(dinv_v, [dlv])
        return 0
    lax.fori_loop(0, ngrp, a_body, 0)

    nch = (cnt + KR - 1) // KR

    def start_gather(c, buf, sem):
        pltpu.make_async_copy(
            hw_hbm.at[srcs_v.at[pl.ds(c * KR, KR)]], buf, sem).start()

    def wait_gather(c, buf, sem):
        pltpu.make_async_copy(
            hw_hbm.at[srcs_v.at[pl.ds(c * KR, KR)]], buf, sem).wait()

    def consume(c, buf):
        def e_body(j, _):
            pos = c * KR + j
            valid = pos < cnt
            dl = jnp.where(valid, dstg_v[pl.ds(pos, 16)][0] - n0, 0)
            ab = jnp.full((16,), alpha_v[pl.ds(pos, 16)][0], jnp.float32)
            vb = jnp.full((16,), valid, jnp.bool_)
            for r in range(8):
                sl = pl.ds(r * 16, 16)
                rowv = buf[j, sl]
                aggv = agg_v[dl, sl]
                newv = jnp.maximum(aggv, ab * rowv)
                agg_v[dl, sl] = jnp.where(vb, newv, aggv)
            return 0
        lax.fori_loop(0, KR, e_body, 0)

    nchm1 = jnp.maximum(nch - 1, 0)
    start_gather(0, rows0_v, sem0)

    def pair_body(p, _):
        c0 = 2 * p
        c1 = jnp.minimum(c0 + 1, nchm1)
        c2 = jnp.minimum(c0 + 2, nchm1)
        start_gather(c1, rows1_v, sem1)
        wait_gather(c0, rows0_v, sem0)
        consume(c0, rows0_v)
        start_gather(c2, rows0_v, sem0)
        wait_gather(c1, rows1_v, sem1)
        consume(c0 + 1, rows1_v)
        return 0
    lax.fori_loop(0, (nch + 1) // 2, pair_body, 0)
    wait_gather(0, rows0_v, sem0)

    def fin_body(d, _):
        for r in range(8):
            sl = pl.ds(r * 16, 16)
            v = agg_v[d, sl]
            elu = jnp.where(v > 0.0, v, jnp.exp(v) - 1.0)
            agg_v[d, sl] = jnp.where(v < -3e38, 0.0, elu)
        return 0
    lax.fori_loop(0, NPT, fin_body, 0)
    pltpu.sync_copy(agg_v, out_hbm.at[pl.ds(n0, NPT)])


@jax.jit
def _sc_edge(hw, s_pad, t_pad, srcb, dstb, cnt):
    mesh = plsc.VectorSubcoreMesh(core_axis_name="c", subcore_axis_name="s")
    f = pl.kernel(
        _sc_edge_body,
        mesh=mesh,
        compiler_params=pltpu.CompilerParams(needs_layout_passes=False),
        out_type=jax.ShapeDtypeStruct((N_PAD, H), jnp.float32),
        scratch_types=[
            pltpu.VMEM((N_PAD,), jnp.float32),
            pltpu.VMEM((NPT,), jnp.float32),
            pltpu.VMEM((NPT,), jnp.float32),
            pltpu.VMEM((16,), jnp.int32),
            pltpu.VMEM((ECAP + 16,), jnp.int32),
            pltpu.VMEM((ECAP + 144,), jnp.int32),
            pltpu.VMEM((ECAP + 144,), jnp.float32),
            pltpu.VMEM((KR, H), jnp.float32),
            pltpu.VMEM((KR, H), jnp.float32),
            pltpu.VMEM((NPT, H), jnp.float32),
            pltpu.SemaphoreType.DMA,
            pltpu.SemaphoreType.DMA,
        ],
    )
    return f(hw, s_pad, t_pad, srcb, dstb, cnt)


def kernel(x, edge_index, Wg, a_src, a_dst, Wz, Uz, bz, Wr, Ur, br,
           Wh, Uh, bh):
    src = edge_index[0]
    dst = edge_index[1]
    src_p = jnp.pad(src, (0, E_PAD - E))
    dst_p = jnp.pad(dst, (0, E_PAD - E), constant_values=N_PAD)
    srcb, dstb, cnt = _sc_bin(src_p, dst_p)

    xz, xr, xh, h = _k0(x, Wz, Wr, Wh, bz, br, bh)
    for _ in range(2):
        hw, s, t = _k1(h, Wg, a_src, a_dst)
        s_pad = jnp.pad(s[:, 0], (0, N_PAD - N))
        t_pad = jnp.pad(t[:, 0], (0, N_PAD - N))
        agg = _sc_edge(hw, s_pad, t_pad, srcb, dstb, cnt)[:N]
        h = _k2(agg, xz, xr, xh, Uz, Ur, Uh)
    return h


# speedup vs baseline: 11.6510x; 1.0875x over previous
import functools

import jax
import jax.numpy as jnp
from jax import lax
from jax.experimental import pallas as pl
from jax.experimental.pallas import tpu as pltpu
from jax.experimental.pallas import tpu_sc as plsc

N = 10000
E = 320000
H = 128
N_BLK = 400

NW = 32
NPT = 320
N_PAD = NW * NPT
ECAP = 12288
KR = 128
CE = 4096
E_PAD = ((E + CE - 1) // CE) * CE
NEG = float("-inf")


def _k0_body(x_ref, wz_ref, wr_ref, wh_ref, bz_ref, br_ref, bh_ref,
             xz_ref, xr_ref, xh_ref, h1_ref):
    x = x_ref[...]
    xz = x @ wz_ref[...] + bz_ref[...]
    xr = x @ wr_ref[...] + br_ref[...]
    xh = x @ wh_ref[...] + bh_ref[...]
    xz_ref[...] = xz
    xr_ref[...] = xr
    xh_ref[...] = xh
    h1_ref[...] = jax.nn.sigmoid(xz) * jnp.tanh(xh)


def _k1_body(h_ref, wg_ref, asrc_ref, adst_ref, hw_ref, s_ref, t_ref):
    hw = h_ref[...] @ wg_ref[...]
    hw_ref[...] = hw
    s_ref[...] = hw @ asrc_ref[...]
    t_ref[...] = hw @ adst_ref[...]


def _k2_body(agg_ref, xz_ref, xr_ref, xh_ref, uz_ref, ur_ref, uh_ref,
             out_ref):
    agg = agg_ref[...]
    z = jax.nn.sigmoid(xz_ref[...] + agg @ uz_ref[...])
    r = jax.nn.sigmoid(xr_ref[...] + agg @ ur_ref[...])
    htil = jnp.tanh(xh_ref[...] + (r * agg) @ uh_ref[...])
    out_ref[...] = (1.0 - z) * agg + z * htil


def _row_spec():
    return pl.BlockSpec((N_BLK, H), lambda i: (i, 0))


def _full_spec(shape):
    return pl.BlockSpec(shape, lambda i: tuple(0 for _ in shape))


_GRID = (N // N_BLK,)


@jax.jit
def _k0(x, wz, wr, wh, bz, br, bh):
    f = pl.pallas_call(
        _k0_body,
        grid=_GRID,
        in_specs=[_row_spec()] + [_full_spec((H, H))] * 3
        + [_full_spec((1, H))] * 3,
        out_specs=[_row_spec()] * 4,
        out_shape=[jax.ShapeDtypeStruct((N, H), jnp.float32)] * 4,
    )
    return f(x, wz, wr, wh, bz.reshape(1, H), br.reshape(1, H),
             bh.reshape(1, H))


@jax.jit
def _k1(h, wg, a_src, a_dst):
    f = pl.pallas_call(
        _k1_body,
        grid=_GRID,
        in_specs=[_row_spec(), _full_spec((H, H)), _full_spec((H, 1)),
                  _full_spec((H, 1))],
        out_specs=[_row_spec(), pl.BlockSpec((N_BLK, 1), lambda i: (i, 0)),
                   pl.BlockSpec((N_BLK, 1), lambda i: (i, 0))],
        out_shape=[jax.ShapeDtypeStruct((N, H), jnp.float32),
                   jax.ShapeDtypeStruct((N, 1), jnp.float32),
                   jax.ShapeDtypeStruct((N, 1), jnp.float32)],
    )
    return f(h, wg, a_src.reshape(H, 1), a_dst.reshape(H, 1))


@jax.jit
def _k2(agg, xz, xr, xh, uz, ur, uh):
    f = pl.pallas_call(
        _k2_body,
        grid=_GRID,
        in_specs=[_row_spec()] * 4 + [_full_spec((H, H))] * 3,
        out_specs=_row_spec(),
        out_shape=jax.ShapeDtypeStruct((N, H), jnp.float32),
    )
    return f(agg, xz, xr, xh, uz, ur, uh)


def _sc_bin_body(src_hbm, dst_hbm, srcb_hbm, dstb_hbm, cnt_hbm,
                 sbuf0, dbuf0, sbuf1, dbuf1, src_loc, dst_loc, cnt16_v,
                 semA, semB):
    wid = lax.axis_index("s") * 2 + lax.axis_index("c")
    n0 = wid * NPT

    def memset_body(i, _):
        src_loc[pl.ds(i * 16, 16)] = jnp.zeros((16,), jnp.int32)
        dst_loc[pl.ds(i * 16, 16)] = jnp.full((16,), N_PAD, jnp.int32)
        return 0
    lax.fori_loop(0, (ECAP + 16) // 16, memset_body, 0)

    def start_pair(c, sb, db, sem):
        pltpu.make_async_copy(src_hbm.at[pl.ds(c * CE, CE)], sb, sem).start()
        pltpu.make_async_copy(dst_hbm.at[pl.ds(c * CE, CE)], db, sem).start()

    def wait_pair(c, sb, db, sem):
        pltpu.make_async_copy(src_hbm.at[pl.ds(c * CE, CE)], sb, sem).wait()
        pltpu.make_async_copy(dst_hbm.at[pl.ds(c * CE, CE)], db, sem).wait()

    def scan(sb, db, cnt):
        def g_body(g, cnt):
            sl = pl.ds(g * 16, 16)
            d16 = db[sl]
            s16 = sb[sl]
            m = (d16 >= n0) & (d16 < n0 + NPT)
            plsc.store_compressed(src_loc.at[pl.ds(cnt, 16)], s16, mask=m)
            plsc.store_compressed(dst_loc.at[pl.ds(cnt, 16)], d16, mask=m)
            return cnt + plsc.all_reduce_population_count(m)[0]
        return lax.fori_loop(0, CE // 16, g_body, cnt)

    start_pair(0, sbuf0, dbuf0, semA)

    def pbody(p, cnt):
        c0 = 2 * p
        start_pair(c0 + 1, sbuf1, dbuf1, semB)
        wait_pair(c0, sbuf0, dbuf0, semA)
        cnt = scan(sbuf0, dbuf0, cnt)
        start_pair(c0 + 2, sbuf0, dbuf0, semA)
        wait_pair(c0 + 1, sbuf1, dbuf1, semB)
        return scan(sbuf1, dbuf1, cnt)
    cnt = lax.fori_loop(0, (E_PAD // CE) // 2, pbody, 0)
    wait_pair(E_PAD // CE - 1, sbuf0, dbuf0, semA)
    cnt = scan(sbuf0, dbuf0, cnt)

    cnt16_v[...] = jnp.full((16,), cnt, jnp.int32)
    pltpu.sync_copy(cnt16_v, cnt_hbm.at[pl.ds(wid * 16, 16)])
    pltpu.sync_copy(src_loc.at[pl.ds(0, ECAP)],
                    srcb_hbm.at[pl.ds(wid * ECAP, ECAP)])
    pltpu.sync_copy(dst_loc.at[pl.ds(0, ECAP)],
                    dstb_hbm.at[pl.ds(wid * ECAP, ECAP)])


@jax.jit
def _sc_bin(src_p, dst_p):
    mesh = plsc.VectorSubcoreMesh(core_axis_name="c", subcore_axis_name="s")
    f = pl.kernel(
        _sc_bin_body,
        mesh=mesh,
        compiler_params=pltpu.CompilerParams(needs_layout_passes=False),
        out_type=[jax.ShapeDtypeStruct((NW * ECAP,), jnp.int32),
                  jax.ShapeDtypeStruct((NW * ECAP,), jnp.int32),
                  jax.ShapeDtypeStruct((NW * 16,), jnp.int32)],
        scratch_types=[
            pltpu.VMEM((CE,), jnp.int32),
            pltpu.VMEM((CE,), jnp.int32),
            pltpu.VMEM((CE,), jnp.int32),
            pltpu.VMEM((CE,), jnp.int32),
            pltpu.VMEM((ECAP + 16,), jnp.int32),
            pltpu.VMEM((ECAP + 16,), jnp.int32),
            pltpu.VMEM((16,), jnp.int32),
            pltpu.SemaphoreType.DMA,
            pltpu.SemaphoreType.DMA,
        ],
    )
    return f(src_p, dst_p)


def _sc_edge_body(hw_hbm, s_hbm, t_hbm, srcb_hbm, dstb_hbm, cnt_hbm, out_hbm,
                  s_v, t_v, dinv_v, cnt16_v, srcs_v, dstg_v, alpha_v,
                  rows0_v, rows1_v, agg_v, sem0, sem1):
    wid = lax.axis_index("s") * 2 + lax.axis_index("c")
    n0 = wid * NPT

    pltpu.sync_copy(s_hbm, s_v)
    pltpu.sync_copy(t_hbm.at[pl.ds(n0, NPT)], t_v)
    pltpu.sync_copy(cnt_hbm.at[pl.ds(wid * 16, 16)], cnt16_v)
    cnt = cnt16_v[...][0]
    pltpu.sync_copy(srcb_hbm.at[pl.ds(wid * ECAP, ECAP)],
                    srcs_v.at[pl.ds(0, ECAP)])
    pltpu.sync_copy(dstb_hbm.at[pl.ds(wid * ECAP, ECAP)],
                    dstg_v.at[pl.ds(0, ECAP)])

    lanes = lax.iota(jnp.int32, 16)
    ngrp = (cnt + 15) // 16

    def zero_body(i, _):
        dinv_v[pl.ds(i * 16, 16)] = jnp.zeros((16,), jnp.float32)
        return 0
    lax.fori_loop(0, NPT // 16, zero_body, 0)

    def agg_init(d, _):
        for r in range(8):
            agg_v[d, pl.ds(r * 16, 16)] = jnp.full((16,), NEG, jnp.float32)
        return 0
    lax.fori_loop(0, NPT, agg_init, 0)

    def p1_body(g, _):
        sl = pl.ds(g * 16, 16)
        valid = (g * 16 + lanes) < cnt
        srcv = srcs_v[sl]
        dlv = jnp.where(valid, dstg_v[sl] - n0, 0)
        sg = plsc.load_gather(s_v, [srcv])
        tg = plsc.load_gather(t_v, [dlv])
        e = sg + tg
        e = jnp.where(e >= 0.0, e, 0.2 * e)
        w = jnp.where(valid, jnp.exp(e), 0.0)
        alpha_v[sl] = w
        plsc.addupdate_scatter(dinv_v, [dlv], w)
        return 0
    lax.fori_loop(0, ngrp, p1_body, 0)

    def inv_body(i, _):
        sl = pl.ds(i * 16, 16)
        dinv_v[sl] = 1.0 / dinv_v[sl]
        return 0
    lax.fori_loop(0, NPT // 16, inv_body, 0)

    def a_body(g, _):
        sl = pl.ds(g * 16, 16)
        valid = (g * 16 + lanes) < cnt
        dlv = jnp.where(valid, dstg_v[sl] - n0, 0)
        alpha_v[sl] = alpha_v[sl] * plsc.load_gather(dinv_v, [dlv])
        return 0
    lax.fori_loop(0, ngrp, a_body, 0)

    nch = (cnt + KR - 1) // KR

    def start_gather(c, buf, sem):
        pltpu.make_async_copy(
            hw_hbm.at[srcs_v.at[pl.ds(c * KR, KR)]], buf, sem).start()

    def wait_gather(c, buf, sem):
        pltpu.make_async_copy(
            hw_hbm.at[srcs_v.at[pl.ds(c * KR, KR)]], buf, sem).wait()

    negv = jnp.full((16,), NEG, jnp.float32)

    def consume(c, buf, carry):
        def grp_body(gq, carry):
            cur, acc = carry
            base = c * KR + gq * 16
            dl16 = dstg_v[pl.ds(base, 16)] - n0
            al16 = alpha_v[pl.ds(base, 16)]
            val16 = ((base + lanes) < cnt).astype(jnp.int32)
            for j in range(16):
                valid = val16[j] != 0
                dl = jnp.where(valid, dl16[j], cur)
                same = dl == cur
                ab = jnp.full((16,), al16[j], jnp.float32)
                vb = jnp.full((16,), valid, jnp.bool_)
                sb = jnp.full((16,), same, jnp.bool_)
                newacc = []
                for r in range(8):
                    rowv = buf[gq * 16 + j, pl.ds(r * 16, 16)]
                    v = jnp.where(vb, ab * rowv, negv)
                    a_r = jnp.where(sb, jnp.maximum(acc[r], v), v)
                    agg_v[dl, pl.ds(r * 16, 16)] = a_r
                    newacc.append(a_r)
                acc = tuple(newacc)
                cur = dl
            return (cur, acc)
        return lax.fori_loop(0, KR // 16, grp_body, carry)

    nchm1 = jnp.maximum(nch - 1, 0)
    start_gather(0, rows0_v, sem0)
    carry0 = (jnp.int32(0), tuple(jnp.full((16,), NEG, jnp.float32)
                                  for _ in range(8)))

    def pair_body(p, carry):
        c0 = 2 * p
        c1 = jnp.minimum(c0 + 1, nchm1)
        c2 = jnp.minimum(c0 + 2, nchm1)
        start_gather(c1, rows1_v, sem1)
        wait_gather(c0, rows0_v, sem0)
        carry = consume(c0, rows0_v, carry)
        start_gather(c2, rows0_v, sem0)
        wait_gather(c1, rows1_v, sem1)
        return consume(c0 + 1, rows1_v, carry)
    lax.fori_loop(0, (nch + 1) // 2, pair_body, carry0)
    wait_gather(0, rows0_v, sem0)

    def fin_body(d, _):
        for r in range(8):
            sl = pl.ds(r * 16, 16)
            v = agg_v[d, sl]
            elu = jnp.where(v > 0.0, v, jnp.exp(v) - 1.0)
            agg_v[d, sl] = jnp.where(v < -3e38, 0.0, elu)
        return 0
    lax.fori_loop(0, NPT, fin_body, 0)
    pltpu.sync_copy(agg_v, out_hbm.at[pl.ds(n0, NPT)])


@jax.jit
def _sc_edge(hw, s_pad, t_pad, srcb, dstb, cnt):
    mesh = plsc.VectorSubcoreMesh(core_axis_name="c", subcore_axis_name="s")
    f = pl.kernel(
        _sc_edge_body,
        mesh=mesh,
        compiler_params=pltpu.CompilerParams(needs_layout_passes=False),
        out_type=jax.ShapeDtypeStruct((N_PAD, H), jnp.float32),
        scratch_types=[
            pltpu.VMEM((N_PAD,), jnp.float32),
            pltpu.VMEM((NPT,), jnp.float32),
            pltpu.VMEM((NPT,), jnp.float32),
            pltpu.VMEM((16,), jnp.int32),
            pltpu.VMEM((ECAP + 16,), jnp.int32),
            pltpu.VMEM((ECAP + 144,), jnp.int32),
            pltpu.VMEM((ECAP + 144,), jnp.float32),
            pltpu.VMEM((KR, H), jnp.float32),
            pltpu.VMEM((KR, H), jnp.float32),
            pltpu.VMEM((NPT, H), jnp.float32),
            pltpu.SemaphoreType.DMA,
            pltpu.SemaphoreType.DMA,
        ],
    )
    return f(hw, s_pad, t_pad, srcb, dstb, cnt)


def kernel(x, edge_index, Wg, a_src, a_dst, Wz, Uz, bz, Wr, Ur, br,
           Wh, Uh, bh):
    src = edge_index[0]
    dst = edge_index[1]
    src_p = jnp.pad(src, (0, E_PAD - E))
    dst_p = jnp.pad(dst, (0, E_PAD - E), constant_values=N_PAD)
    srcb, dstb, cnt = _sc_bin(src_p, dst_p)

    xz, xr, xh, h = _k0(x, Wz, Wr, Wh, bz, br, bh)
    for _ in range(2):
        hw, s, t = _k1(h, Wg, a_src, a_dst)
        s_pad = jnp.pad(s[:, 0], (0, N_PAD - N))
        t_pad = jnp.pad(t[:, 0], (0, N_PAD - N))
        agg = _sc_edge(hw, s_pad, t_pad, srcb, dstb, cnt)[:N]
        h = _k2(agg, xz, xr, xh, Uz, Ur, Uh)
    return h


# speedup vs baseline: 12.2166x; 1.0485x over previous
import functools

import jax
import jax.numpy as jnp
from jax import lax
from jax.experimental import pallas as pl
from jax.experimental.pallas import tpu as pltpu
from jax.experimental.pallas import tpu_sc as plsc

N = 10000
E = 320000
H = 128
N_BLK = 400

NW = 32
NPT = 320
N_PAD = NW * NPT
ECAP = 12288
KR = 128
CE = 4096
E_PAD = ((E + CE - 1) // CE) * CE
NEG = float("-inf")


def _k0_body(x_ref, wz_ref, wr_ref, wh_ref, bz_ref, br_ref, bh_ref,
             xz_ref, xr_ref, xh_ref, h1_ref):
    x = x_ref[...]
    xz = x @ wz_ref[...] + bz_ref[...]
    xr = x @ wr_ref[...] + br_ref[...]
    xh = x @ wh_ref[...] + bh_ref[...]
    xz_ref[...] = xz
    xr_ref[...] = xr
    xh_ref[...] = xh
    h1_ref[...] = jax.nn.sigmoid(xz) * jnp.tanh(xh)


def _k1_body(h_ref, wg_ref, asrc_ref, adst_ref, hw_ref, s_ref, t_ref):
    hw = h_ref[...] @ wg_ref[...]
    hw_ref[...] = hw
    s_ref[...] = hw @ asrc_ref[...]
    t_ref[...] = hw @ adst_ref[...]


def _k2_body(agg_ref, xz_ref, xr_ref, xh_ref, uz_ref, ur_ref, uh_ref,
             out_ref):
    agg = agg_ref[...]
    z = jax.nn.sigmoid(xz_ref[...] + agg @ uz_ref[...])
    r = jax.nn.sigmoid(xr_ref[...] + agg @ ur_ref[...])
    htil = jnp.tanh(xh_ref[...] + (r * agg) @ uh_ref[...])
    out_ref[...] = (1.0 - z) * agg + z * htil


def _row_spec():
    return pl.BlockSpec((N_BLK, H), lambda i: (i, 0))


def _full_spec(shape):
    return pl.BlockSpec(shape, lambda i: tuple(0 for _ in shape))


_GRID = (N // N_BLK,)


@jax.jit
def _k0(x, wz, wr, wh, bz, br, bh):
    f = pl.pallas_call(
        _k0_body,
        grid=_GRID,
        in_specs=[_row_spec()] + [_full_spec((H, H))] * 3
        + [_full_spec((1, H))] * 3,
        out_specs=[_row_spec()] * 4,
        out_shape=[jax.ShapeDtypeStruct((N, H), jnp.float32)] * 4,
    )
    return f(x, wz, wr, wh, bz.reshape(1, H), br.reshape(1, H),
             bh.reshape(1, H))


@jax.jit
def _k1(h, wg, a_src, a_dst):
    f = pl.pallas_call(
        _k1_body,
        grid=_GRID,
        in_specs=[_row_spec(), _full_spec((H, H)), _full_spec((H, 1)),
                  _full_spec((H, 1))],
        out_specs=[_row_spec(), pl.BlockSpec((N_BLK, 1), lambda i: (i, 0)),
                   pl.BlockSpec((N_BLK, 1), lambda i: (i, 0))],
        out_shape=[jax.ShapeDtypeStruct((N, H), jnp.float32),
                   jax.ShapeDtypeStruct((N, 1), jnp.float32),
                   jax.ShapeDtypeStruct((N, 1), jnp.float32)],
    )
    return f(h, wg, a_src.reshape(H, 1), a_dst.reshape(H, 1))


@jax.jit
def _k2(agg, xz, xr, xh, uz, ur, uh):
    f = pl.pallas_call(
        _k2_body,
        grid=_GRID,
        in_specs=[_row_spec()] * 4 + [_full_spec((H, H))] * 3,
        out_specs=_row_spec(),
        out_shape=jax.ShapeDtypeStruct((N, H), jnp.float32),
    )
    return f(agg, xz, xr, xh, uz, ur, uh)


def _sc_bin_body(src_hbm, dst_hbm, srcb_hbm, dstb_hbm, cnt_hbm,
                 sbuf0, dbuf0, sbuf1, dbuf1, src_loc, dst_loc, cnt16_v,
                 semA, semB):
    wid = lax.axis_index("s") * 2 + lax.axis_index("c")
    n0 = wid * NPT

    def memset_body(i, _):
        src_loc[pl.ds(i * 16, 16)] = jnp.zeros((16,), jnp.int32)
        dst_loc[pl.ds(i * 16, 16)] = jnp.full((16,), N_PAD, jnp.int32)
        return 0
    lax.fori_loop(0, (ECAP + 16) // 16, memset_body, 0)

    def start_pair(c, sb, db, sem):
        pltpu.make_async_copy(src_hbm.at[pl.ds(c * CE, CE)], sb, sem).start()
        pltpu.make_async_copy(dst_hbm.at[pl.ds(c * CE, CE)], db, sem).start()

    def wait_pair(c, sb, db, sem):
        pltpu.make_async_copy(src_hbm.at[pl.ds(c * CE, CE)], sb, sem).wait()
        pltpu.make_async_copy(dst_hbm.at[pl.ds(c * CE, CE)], db, sem).wait()

    def scan(sb, db, cnt):
        def g_body(g, cnt):
            sl = pl.ds(g * 16, 16)
            d16 = db[sl]
            s16 = sb[sl]
            m = (d16 >= n0) & (d16 < n0 + NPT)
            plsc.store_compressed(src_loc.at[pl.ds(cnt, 16)], s16, mask=m)
            plsc.store_compressed(dst_loc.at[pl.ds(cnt, 16)], d16, mask=m)
            return cnt + plsc.all_reduce_population_count(m)[0]
        return lax.fori_loop(0, CE // 16, g_body, cnt)

    start_pair(0, sbuf0, dbuf0, semA)

    def pbody(p, cnt):
        c0 = 2 * p
        start_pair(c0 + 1, sbuf1, dbuf1, semB)
        wait_pair(c0, sbuf0, dbuf0, semA)
        cnt = scan(sbuf0, dbuf0, cnt)
        start_pair(c0 + 2, sbuf0, dbuf0, semA)
        wait_pair(c0 + 1, sbuf1, dbuf1, semB)
        return scan(sbuf1, dbuf1, cnt)
    cnt = lax.fori_loop(0, (E_PAD // CE) // 2, pbody, 0)
    wait_pair(E_PAD // CE - 1, sbuf0, dbuf0, semA)
    cnt = scan(sbuf0, dbuf0, cnt)

    cnt16_v[...] = jnp.full((16,), cnt, jnp.int32)
    pltpu.sync_copy(cnt16_v, cnt_hbm.at[pl.ds(wid * 16, 16)])
    pltpu.sync_copy(src_loc.at[pl.ds(0, ECAP)],
                    srcb_hbm.at[pl.ds(wid * ECAP, ECAP)])
    pltpu.sync_copy(dst_loc.at[pl.ds(0, ECAP)],
                    dstb_hbm.at[pl.ds(wid * ECAP, ECAP)])


@jax.jit
def _sc_bin(src_p, dst_p):
    mesh = plsc.VectorSubcoreMesh(core_axis_name="c", subcore_axis_name="s")
    f = pl.kernel(
        _sc_bin_body,
        mesh=mesh,
        compiler_params=pltpu.CompilerParams(needs_layout_passes=False),
        out_type=[jax.ShapeDtypeStruct((NW * ECAP,), jnp.int32),
                  jax.ShapeDtypeStruct((NW * ECAP,), jnp.int32),
                  jax.ShapeDtypeStruct((NW * 16,), jnp.int32)],
        scratch_types=[
            pltpu.VMEM((CE,), jnp.int32),
            pltpu.VMEM((CE,), jnp.int32),
            pltpu.VMEM((CE,), jnp.int32),
            pltpu.VMEM((CE,), jnp.int32),
            pltpu.VMEM((ECAP + 16,), jnp.int32),
            pltpu.VMEM((ECAP + 16,), jnp.int32),
            pltpu.VMEM((16,), jnp.int32),
            pltpu.SemaphoreType.DMA,
            pltpu.SemaphoreType.DMA,
        ],
    )
    return f(src_p, dst_p)


def _sc_edge_body(hw_hbm, s_hbm, t_hbm, srcb_hbm, dstb_hbm, cnt_hbm, out_hbm,
                  s_v, t_v, dinv_v, cnt16_v, srcs_v, dstg_v, alpha_v,
                  rows0_v, rows1_v, agg_v, sem0, sem1):
    wid = lax.axis_index("s") * 2 + lax.axis_index("c")
    n0 = wid * NPT

    pltpu.sync_copy(s_hbm, s_v)
    pltpu.sync_copy(t_hbm.at[pl.ds(n0, NPT)], t_v)
    pltpu.sync_copy(cnt_hbm.at[pl.ds(wid * 16, 16)], cnt16_v)
    cnt = cnt16_v[...][0]
    pltpu.sync_copy(srcb_hbm.at[pl.ds(wid * ECAP, ECAP)],
                    srcs_v.at[pl.ds(0, ECAP)])
    pltpu.sync_copy(dstb_hbm.at[pl.ds(wid * ECAP, ECAP)],
                    dstg_v.at[pl.ds(0, ECAP)])

    lanes = lax.iota(jnp.int32, 16)
    ngrp = (cnt + 15) // 16

    def zero_body(i, _):
        dinv_v[pl.ds(i * 16, 16)] = jnp.zeros((16,), jnp.float32)
        return 0
    lax.fori_loop(0, NPT // 16, zero_body, 0)

    def agg_init(d, _):
        for r in range(8):
            agg_v[d, pl.ds(r * 16, 16)] = jnp.full((16,), NEG, jnp.float32)
        return 0
    lax.fori_loop(0, NPT, agg_init, 0)

    def p1_body(g, _):
        sl = pl.ds(g * 16, 16)
        valid = (g * 16 + lanes) < cnt
        srcv = srcs_v[sl]
        dlv = jnp.where(valid, dstg_v[sl] - n0, 0)
        sg = plsc.load_gather(s_v, [srcv])
        tg = plsc.load_gather(t_v, [dlv])
        e = sg + tg
        e = jnp.where(e >= 0.0, e, 0.2 * e)
        w = jnp.where(valid, jnp.exp(e), 0.0)
        alpha_v[sl] = w
        plsc.addupdate_scatter(dinv_v, [dlv], w)
        return 0
    lax.fori_loop(0, ngrp, p1_body, 0)

    def inv_body(i, _):
        sl = pl.ds(i * 16, 16)
        dinv_v[sl] = 1.0 / dinv_v[sl]
        return 0
    lax.fori_loop(0, NPT // 16, inv_body, 0)

    def a_body(g, _):
        sl = pl.ds(g * 16, 16)
        valid = (g * 16 + lanes) < cnt
        dlv = jnp.where(valid, dstg_v[sl] - n0, 0)
        alpha_v[sl] = alpha_v[sl] * plsc.load_gather(dinv_v, [dlv])
        return 0
    lax.fori_loop(0, ngrp, a_body, 0)

    nch = (cnt + KR - 1) // KR

    def start_gather(c, buf, sem):
        pltpu.make_async_copy(
            hw_hbm.at[srcs_v.at[pl.ds(c * KR, KR)]], buf, sem).start()

    def wait_gather(c, buf, sem):
        pltpu.make_async_copy(
            hw_hbm.at[srcs_v.at[pl.ds(c * KR, KR)]], buf, sem).wait()

    def consume(c, buf):
        def grp_body(gq, _):
            base = c * KR + gq * 16
            dl16 = dstg_v[pl.ds(base, 16)] - n0
            al16 = alpha_v[pl.ds(base, 16)]
            val16 = ((base + lanes) < cnt).astype(jnp.int32)
            for j in range(16):
                valid = val16[j] != 0
                dl = jnp.where(valid, dl16[j], 0)
                ab = jnp.full((16,), al16[j], jnp.float32)
                vb = jnp.full((16,), valid, jnp.bool_)
                for r in range(8):
                    sl = pl.ds(r * 16, 16)
                    rowv = buf[gq * 16 + j, sl]
                    aggv = agg_v[dl, sl]
                    newv = jnp.maximum(aggv, ab * rowv)
                    agg_v[dl, sl] = jnp.where(vb, newv, aggv)
            return 0
        lax.fori_loop(0, KR // 16, grp_body, 0)

    nchm1 = jnp.maximum(nch - 1, 0)
    start_gather(0, rows0_v, sem0)

    def pair_body(p, _):
        c0 = 2 * p
        c1 = jnp.minimum(c0 + 1, nchm1)
        c2 = jnp.minimum(c0 + 2, nchm1)
        start_gather(c1, rows1_v, sem1)
        wait_gather(c0, rows0_v, sem0)
        consume(c0, rows0_v)
        start_gather(c2, rows0_v, sem0)
        wait_gather(c1, rows1_v, sem1)
        consume(c0 + 1, rows1_v)
        return 0
    lax.fori_loop(0, (nch + 1) // 2, pair_body, 0)
    wait_gather(0, rows0_v, sem0)

    def fin_body(d, _):
        for r in range(8):
            sl = pl.ds(r * 16, 16)
            v = agg_v[d, sl]
            elu = jnp.where(v > 0.0, v, jnp.exp(v) - 1.0)
            agg_v[d, sl] = jnp.where(v < -3e38, 0.0, elu)
        return 0
    lax.fori_loop(0, NPT, fin_body, 0)
    pltpu.sync_copy(agg_v, out_hbm.at[pl.ds(n0, NPT)])


@jax.jit
def _sc_edge(hw, s_pad, t_pad, srcb, dstb, cnt):
    mesh = plsc.VectorSubcoreMesh(core_axis_name="c", subcore_axis_name="s")
    f = pl.kernel(
        _sc_edge_body,
        mesh=mesh,
        compiler_params=pltpu.CompilerParams(needs_layout_passes=False),
        out_type=jax.ShapeDtypeStruct((N_PAD, H), jnp.float32),
        scratch_types=[
            pltpu.VMEM((N_PAD,), jnp.float32),
            pltpu.VMEM((NPT,), jnp.float32),
            pltpu.VMEM((NPT,), jnp.float32),
            pltpu.VMEM((16,), jnp.int32),
            pltpu.VMEM((ECAP + 16,), jnp.int32),
            pltpu.VMEM((ECAP + 144,), jnp.int32),
            pltpu.VMEM((ECAP + 144,), jnp.float32),
            pltpu.VMEM((KR, H), jnp.float32),
            pltpu.VMEM((KR, H), jnp.float32),
            pltpu.VMEM((NPT, H), jnp.float32),
            pltpu.SemaphoreType.DMA,
            pltpu.SemaphoreType.DMA,
        ],
    )
    return f(hw, s_pad, t_pad, srcb, dstb, cnt)


def kernel(x, edge_index, Wg, a_src, a_dst, Wz, Uz, bz, Wr, Ur, br,
           Wh, Uh, bh):
    src = edge_index[0]
    dst = edge_index[1]
    src_p = jnp.pad(src, (0, E_PAD - E))
    dst_p = jnp.pad(dst, (0, E_PAD - E), constant_values=N_PAD)
    srcb, dstb, cnt = _sc_bin(src_p, dst_p)

    xz, xr, xh, h = _k0(x, Wz, Wr, Wh, bz, br, bh)
    for _ in range(2):
        hw, s, t = _k1(h, Wg, a_src, a_dst)
        s_pad = jnp.pad(s[:, 0], (0, N_PAD - N))
        t_pad = jnp.pad(t[:, 0], (0, N_PAD - N))
        agg = _sc_edge(hw, s_pad, t_pad, srcb, dstb, cnt)[:N]
        h = _k2(agg, xz, xr, xh, Uz, Ur, Uh)
    return h
